# baseline (speedup 1.0000x reference)
import functools

import jax
import jax.numpy as jnp
from jax import lax
from jax.experimental import pallas as pl
from jax.experimental.pallas import tpu as pltpu
from jax.experimental.pallas import tpu_sc as plsc

B = 1024
L = 50
EMB = 64
PHOTO_DIM = 1024
N = B * L

_NC = 2
_NS = 16
_NW = _NC * _NS
_RPW = N // _NW
_WCH = 800
_SCH = 200

_SEG_REP = 1024


def _sc_mesh():
    return plsc.VectorSubcoreMesh(
        core_axis_name="c", subcore_axis_name="s",
        num_cores=_NC, num_subcores=_NS)


def _sc_gather_seg(seg_flat, segtab128):
    @functools.partial(
        pl.kernel,
        out_type=jax.ShapeDtypeStruct((N, 128), jnp.float32),
        mesh=_sc_mesh(),
        scratch_types=[
            pltpu.VMEM((_RPW,), jnp.int32),
            pltpu.VMEM((_SCH, 128), jnp.float32),
            pltpu.SemaphoreType.DMA,
        ],
        compiler_params=pltpu.CompilerParams(use_tc_tiling_on_sc=True),
    )
    def seg_k(seg_hbm, stab_hbm, sout_hbm, sidx_v, segrows_v, ssem):
        wid = lax.axis_index("s") * _NC + lax.axis_index("c")
        base = wid * _RPW
        pltpu.sync_copy(seg_hbm.at[pl.ds(base, _RPW)], sidx_v)
        for j in range(_RPW // _SCH):
            pltpu.async_copy(
                stab_hbm.at[sidx_v.at[pl.ds(j * _SCH, _SCH)]],
                segrows_v, ssem).wait()
            pltpu.sync_copy(segrows_v,
                            sout_hbm.at[pl.ds(base + j * _SCH, _SCH)])

    return seg_k(seg_flat, segtab128)


def _sc_gather_word(src_flat, word_table):
    @functools.partial(
        pl.kernel,
        out_type=jax.ShapeDtypeStruct((N, EMB), jnp.float32),
        mesh=_sc_mesh(),
        scratch_types=[
            pltpu.VMEM((_RPW,), jnp.int32),
            pltpu.VMEM((_WCH, EMB), jnp.float32),
            pltpu.SemaphoreType.DMA,
        ],
        compiler_params=pltpu.CompilerParams(use_tc_tiling_on_sc=True),
    )
    def word_k(src_hbm, wtab_hbm, wout_hbm, idx_v, wrows_v, wsem):
        wid = lax.axis_index("s") * _NC + lax.axis_index("c")
        base = wid * _RPW
        pltpu.sync_copy(src_hbm.at[pl.ds(base, _RPW)], idx_v)

        def drain16():
            pltpu.make_async_copy(
                wtab_hbm.at[0, pl.ds(0, 16)],
                wrows_v.at[pl.ds(0, 16)], wsem).wait()

        for c in range(_RPW // _WCH):
            def body(g, _, c=c):
                ivec = idx_v[pl.ds(c * _WCH + g * 16, 16)]
                for k in range(16):
                    i = ivec[k]
                    i0 = i // 500000
                    i1 = i - i0 * 500000
                    pltpu.make_async_copy(
                        wtab_hbm.at[i0, pl.ds(i1, 1)],
                        wrows_v.at[pl.ds(g * 16 + k, 1)], wsem).start()
                lax.cond(g > 2, drain16, lambda: None)
                return 0
            lax.fori_loop(0, _WCH // 16, body, 0, unroll=False)
            for _ in range(3):
                drain16()
            pltpu.sync_copy(wrows_v,
                            wout_hbm.at[pl.ds(base + c * _WCH, _WCH)])

    return word_k(src_flat, word_table.reshape(2, 500000, EMB))


def _tc1_body(vis_ref, W_ref, bvec_ref, pos_ref, out_ref):
    x = jnp.dot(vis_ref[0], W_ref[...], preferred_element_type=jnp.float32)
    out_ref[0] = x + bvec_ref[...] + pos_ref[0]


def _tc1(vis_t, W_vis, b_vis, pos_table):
    return pl.pallas_call(
        _tc1_body,
        grid=(L,),
        in_specs=[
            pl.BlockSpec((1, B, PHOTO_DIM), lambda l: (l, 0, 0)),
            pl.BlockSpec((PHOTO_DIM, EMB), lambda l: (0, 0)),
            pl.BlockSpec((1, EMB), lambda l: (0, 0)),
            pl.BlockSpec((1, 1, EMB), lambda l: (l, 0, 0)),
        ],
        out_specs=pl.BlockSpec((1, B, EMB), lambda l: (l, 0, 0)),
        out_shape=jax.ShapeDtypeStruct((L, B, EMB), jnp.float32),
        compiler_params=pltpu.CompilerParams(
            dimension_semantics=("arbitrary",)),
    )(vis_t, W_vis, b_vis, pos_table.reshape(-1, 1, EMB))


def _tc2_body(part_ref, word_ref, seg_ref, gam_ref, bet_ref, out_ref):
    total = part_ref[0] + word_ref[0] + seg_ref[0][:, :EMB]
    mean = jnp.mean(total, axis=-1, keepdims=True)
    cent = total - mean
    var = jnp.mean(cent * cent, axis=-1, keepdims=True)
    out_ref[0] = cent * lax.rsqrt(var + 1e-6) * gam_ref[...] + bet_ref[...]


def _tc2(part, word3, seg3, gamma, beta):
    return pl.pallas_call(
        _tc2_body,
        grid=(L,),
        in_specs=[
            pl.BlockSpec((1, B, EMB), lambda l: (l, 0, 0)),
            pl.BlockSpec((1, B, EMB), lambda l: (l, 0, 0)),
            pl.BlockSpec((1, B, 128), lambda l: (l, 0, 0)),
            pl.BlockSpec((1, EMB), lambda l: (0, 0)),
            pl.BlockSpec((1, EMB), lambda l: (0, 0)),
        ],
        out_specs=pl.BlockSpec((1, B, EMB), lambda l: (l, 0, 0)),
        out_shape=jax.ShapeDtypeStruct((L, B, EMB), jnp.float32),
        compiler_params=pltpu.CompilerParams(
            dimension_semantics=("arbitrary",)),
    )(part, word3, seg3, gamma, beta)


def kernel(visual, src, seg, word_table, pos_table, seg_table,
           W_vis, b_vis, ln_gamma, ln_beta):
    vis_t = jnp.transpose(visual, (1, 0, 2))
    src_flat = jnp.transpose(src).reshape(N)
    seg_flat = (jnp.transpose(seg).reshape(N)
                + 3 * (jnp.arange(N, dtype=jnp.int32) & (_SEG_REP - 1)))
    segtab128 = jnp.zeros((3 * _SEG_REP, 128), jnp.float32).at[:, :EMB].set(
        jnp.tile(seg_table, (_SEG_REP, 1)))

    part = _tc1(vis_t, W_vis, b_vis.reshape(1, EMB), pos_table)
    segemb = _sc_gather_seg(seg_flat, segtab128)
    word = _sc_gather_word(src_flat, word_table)
    out_t = _tc2(part, word.reshape(L, B, EMB), segemb.reshape(L, B, 128),
                 ln_gamma.reshape(1, EMB), ln_beta.reshape(1, EMB))
    return jnp.transpose(out_t, (1, 0, 2))


# speedup vs baseline: 1.0011x; 1.0011x over previous
import functools

import jax
import jax.numpy as jnp
from jax import lax
from jax.experimental import pallas as pl
from jax.experimental.pallas import tpu as pltpu
from jax.experimental.pallas import tpu_sc as plsc

B = 1024
L = 50
EMB = 64
PHOTO_DIM = 1024
N = B * L

_NC = 2
_NS = 16
_NW = _NC * _NS
_RPW = N // _NW
_WCH = 800
_SCH = 200

_SEG_REP = 1024


def _sc_mesh():
    return plsc.VectorSubcoreMesh(
        core_axis_name="c", subcore_axis_name="s",
        num_cores=_NC, num_subcores=_NS)


def _sc_gather_seg(seg_flat, segtab128):
    @functools.partial(
        pl.kernel,
        out_type=jax.ShapeDtypeStruct((N, 128), jnp.float32),
        mesh=_sc_mesh(),
        scratch_types=[
            pltpu.VMEM((_RPW,), jnp.int32),
            pltpu.VMEM((_SCH, 128), jnp.float32),
            pltpu.SemaphoreType.DMA,
        ],
        compiler_params=pltpu.CompilerParams(use_tc_tiling_on_sc=True),
    )
    def seg_k(seg_hbm, stab_hbm, sout_hbm, sidx_v, segrows_v, ssem):
        wid = lax.axis_index("s") * _NC + lax.axis_index("c")
        base = wid * _RPW
        pltpu.sync_copy(seg_hbm.at[pl.ds(base, _RPW)], sidx_v)
        for j in range(_RPW // _SCH):
            pltpu.async_copy(
                stab_hbm.at[sidx_v.at[pl.ds(j * _SCH, _SCH)]],
                segrows_v, ssem).wait()
            pltpu.sync_copy(segrows_v,
                            sout_hbm.at[pl.ds(base + j * _SCH, _SCH)])

    return seg_k(seg_flat, segtab128)


def _sc_gather_word(src_flat, word_table):
    @functools.partial(
        pl.kernel,
        out_type=jax.ShapeDtypeStruct((N, EMB), jnp.float32),
        mesh=_sc_mesh(),
        scratch_types=[
            pltpu.VMEM((_RPW,), jnp.int32),
            pltpu.VMEM((_WCH, EMB), jnp.float32),
            pltpu.SemaphoreType.DMA,
        ],
        compiler_params=pltpu.CompilerParams(use_tc_tiling_on_sc=True),
    )
    def word_k(src_hbm, wtab_hbm, wout_hbm, idx_v, wrows_v, wsem):
        wid = lax.axis_index("s") * _NC + lax.axis_index("c")
        base = wid * _RPW
        pltpu.sync_copy(src_hbm.at[pl.ds(base, _RPW)], idx_v)

        def drain16():
            pltpu.make_async_copy(
                wtab_hbm.at[0, pl.ds(0, 16)],
                wrows_v.at[pl.ds(0, 16)], wsem).wait()

        for c in range(_RPW // _WCH):
            def body(g, _, c=c):
                ivec = idx_v[pl.ds(c * _WCH + g * 16, 16)]
                for k in range(16):
                    i = ivec[k]
                    i0 = i // 500000
                    i1 = i - i0 * 500000
                    pltpu.make_async_copy(
                        wtab_hbm.at[i0, pl.ds(i1, 1)],
                        wrows_v.at[pl.ds(g * 16 + k, 1)], wsem).start()
                lax.cond(g > 2, drain16, lambda: None)
                return 0
            lax.fori_loop(0, _WCH // 16, body, 0, unroll=False)
            for _ in range(3):
                drain16()
            pltpu.sync_copy(wrows_v,
                            wout_hbm.at[pl.ds(base + c * _WCH, _WCH)])

    return word_k(src_flat, word_table.reshape(2, 500000, EMB))


def _tc1_body(vis_ref, W_ref, bvec_ref, pos_ref, out_ref):
    x = jnp.dot(vis_ref[0], W_ref[...], preferred_element_type=jnp.float32)
    out_ref[0] = x + bvec_ref[...] + pos_ref[0]


def _tc1(vis_t, W_vis, b_vis, pos_table):
    return pl.pallas_call(
        _tc1_body,
        grid=(L,),
        in_specs=[
            pl.BlockSpec((1, B, PHOTO_DIM), lambda l: (l, 0, 0)),
            pl.BlockSpec((PHOTO_DIM, EMB), lambda l: (0, 0)),
            pl.BlockSpec((1, EMB), lambda l: (0, 0)),
            pl.BlockSpec((1, 1, EMB), lambda l: (l, 0, 0)),
        ],
        out_specs=pl.BlockSpec((1, B, EMB), lambda l: (l, 0, 0)),
        out_shape=jax.ShapeDtypeStruct((L, B, EMB), jnp.float32),
        compiler_params=pltpu.CompilerParams(
            dimension_semantics=("arbitrary",)),
    )(vis_t, W_vis, b_vis, pos_table.reshape(-1, 1, EMB))


def _tc2_body(part_ref, word_ref, seg_ref, gam_ref, bet_ref, out_ref):
    total = part_ref[0] + word_ref[0] + seg_ref[0][:, :EMB]
    mean = jnp.mean(total, axis=-1, keepdims=True)
    cent = total - mean
    var = jnp.mean(cent * cent, axis=-1, keepdims=True)
    out_ref[0] = cent * lax.rsqrt(var + 1e-6) * gam_ref[...] + bet_ref[...]


def _tc2(part, word3, seg3, gamma, beta):
    return pl.pallas_call(
        _tc2_body,
        grid=(L,),
        in_specs=[
            pl.BlockSpec((1, B, EMB), lambda l: (l, 0, 0)),
            pl.BlockSpec((1, B, EMB), lambda l: (l, 0, 0)),
            pl.BlockSpec((1, B, 128), lambda l: (l, 0, 0)),
            pl.BlockSpec((1, EMB), lambda l: (0, 0)),
            pl.BlockSpec((1, EMB), lambda l: (0, 0)),
        ],
        out_specs=pl.BlockSpec((1, B, EMB), lambda l: (l, 0, 0)),
        out_shape=jax.ShapeDtypeStruct((L, B, EMB), jnp.float32),
        compiler_params=pltpu.CompilerParams(
            dimension_semantics=("arbitrary",)),
    )(part, word3, seg3, gamma, beta)


def kernel(visual, src, seg, word_table, pos_table, seg_table,
           W_vis, b_vis, ln_gamma, ln_beta):
    vis_t = jnp.transpose(visual, (1, 0, 2))
    src_flat = jnp.transpose(src).reshape(N)
    seg_flat = (jnp.transpose(seg).reshape(N)
                + 3 * (jnp.arange(N, dtype=jnp.int32) & (_SEG_REP - 1)))
    segtab128 = jnp.zeros((3 * _SEG_REP, 128), jnp.float32).at[:, :EMB].set(
        jnp.tile(seg_table, (_SEG_REP, 1)))

    part = _tc1(vis_t, W_vis, b_vis.reshape(1, EMB), pos_table)
    segemb = _sc_gather_seg(seg_flat, segtab128)
    pin = (part[0, 0, :1] * 0.0).astype(jnp.int32)
    word = _sc_gather_word(src_flat + pin, word_table)
    out_t = _tc2(part, word.reshape(L, B, EMB), segemb.reshape(L, B, 128),
                 ln_gamma.reshape(1, EMB), ln_beta.reshape(1, EMB))
    return jnp.transpose(out_t, (1, 0, 2))
